# Initial kernel scaffold; baseline (speedup 1.0000x reference)
import jax
import jax.numpy as jnp
from jax.experimental import pallas as pl


def kernel(xyzs, Wec1, bec1, Wec2, bec2, Wec3, bec3, Wdc1, bdc1, Wdc2, bdc2, Wdc3, bdc3, Wfp3, bfp3, Wfp2, bfp2, Wfp1, bfp1, Wm1, bm1, Wm2, bm2):
    raise NotImplementedError("write your pallas kernel here")


# speedup vs baseline: 11.2290x; 11.2290x over previous
import functools

import jax
import jax.numpy as jnp
from jax.experimental import pallas as pl

RADIUS = 4.0
NS = 4

_R1SQ = float((RADIUS + 1e-06) ** 2)
_R2SQ = float((2 * RADIUS + 1e-06) ** 2)
_R3SQ = float((3 * RADIUS + 1e-06) ** 2)
_RQ2SQ = float((2 * RADIUS / 4 + 1e-06) ** 2)
_RQ3SQ = float((4 * RADIUS / 4 + 1e-06) ** 2)

_BIG = 1e30


def _fps(x, y, z, npoint):
    B, N = x.shape
    lane = jax.lax.broadcasted_iota(jnp.int32, (B, N), 1)
    lane_o = jax.lax.broadcasted_iota(jnp.int32, (B, npoint), 1)

    def body(i, st):
        dists, far, ox, oy, oz = st
        oh = (lane == far).astype(x.dtype)
        cx = jnp.sum(x * oh, axis=1, keepdims=True)
        cy = jnp.sum(y * oh, axis=1, keepdims=True)
        cz = jnp.sum(z * oh, axis=1, keepdims=True)
        sel = (lane_o == i).astype(x.dtype)
        ox = ox + cx * sel
        oy = oy + cy * sel
        oz = oz + cz * sel
        d = (x - cx) ** 2 + (y - cy) ** 2 + (z - cz) ** 2
        dists = jnp.minimum(dists, d)
        m = jnp.max(dists, axis=1, keepdims=True)
        far = jnp.min(jnp.where(dists == m, lane, N), axis=1, keepdims=True)
        return dists, far, ox, oy, oz

    init = (
        jnp.full((B, N), 1e10, x.dtype),
        jnp.zeros((B, 1), jnp.int32),
        jnp.zeros((B, npoint), x.dtype),
        jnp.zeros((B, npoint), x.dtype),
        jnp.zeros((B, npoint), x.dtype),
    )
    _, _, ox, oy, oz = jax.lax.fori_loop(0, npoint, body, init)
    return ox, oy, oz


def _pair_d2(qT, sT):
    ones = jnp.ones((3, 1), qT.dtype)
    qq = jnp.einsum('cq,co->qo', qT * qT, ones,
                    preferred_element_type=jnp.float32)
    ss = jnp.sum(sT * sT, axis=0, keepdims=True)
    cross = jnp.einsum('cq,cs->qs', qT, sT,
                       preferred_element_type=jnp.float32)
    return qq + ss - 2.0 * cross


def _min_extract(d2, lanes, Ss):
    m = jnp.min(d2, axis=1, keepdims=True)
    idx = jnp.min(jnp.where(d2 == m, lanes, Ss), axis=1, keepdims=True)
    sel = lanes == idx
    return m, sel


def _knn_sel(qT, sT, K, r2):
    Ss = sT.shape[1]
    d2 = _pair_d2(qT, sT)
    lanes = jax.lax.broadcasted_iota(jnp.int32, d2.shape, 1)
    Gs = []
    G0 = None
    for k in range(K):
        m, sel = _min_extract(d2, lanes, Ss)
        G = sel.astype(qT.dtype)
        if k == 0:
            G0 = G
        else:
            G = jnp.where(m > r2, G0, G)
        Gs.append(G)
        d2 = jnp.where(sel, _BIG, d2)
    return Gs


def _gather(feats, G):
    return jnp.einsum('cs,qs->cq', feats, G,
                      preferred_element_type=jnp.float32)


def _lstm_cell_b(qT, sT, h, c, fT, W, b2, K, r2, out_ch):
    Gs = _knn_sel(qT, sT, K, r2)
    C = h.shape[0]
    src = jnp.concatenate([sT, h, c], axis=0)
    hm = cm = None
    for G in Gs:
        gath = _gather(src, G)
        dispT = gath[0:3] - qT
        g_h = gath[3:3 + C]
        g_c = gath[3 + C:3 + 2 * C]
        parts = [dispT] + ([fT] if fT is not None else []) + [g_h]
        ginT = jnp.concatenate(parts, axis=0)
        gates = jnp.dot(W, ginT, preferred_element_type=jnp.float32) + b2
        i_g = jax.nn.sigmoid(gates[0:out_ch])
        f_g = jax.nn.sigmoid(gates[out_ch:2 * out_ch])
        o_g = jax.nn.sigmoid(gates[2 * out_ch:3 * out_ch])
        g_g = jnp.tanh(gates[3 * out_ch:4 * out_ch])
        c_nb = f_g * g_c + i_g * g_g
        h_nb = o_g * jnp.tanh(c_nb)
        hm = h_nb if hm is None else jnp.maximum(hm, h_nb)
        cm = c_nb if cm is None else jnp.maximum(cm, c_nb)
    return hm, cm


def _qgroup_b(qT, sT, feats, K, r2):
    Gs = _knn_sel(qT, sT, K, r2)
    out = None
    for G in Gs:
        g = _gather(feats, G)
        out = g if out is None else jnp.maximum(out, g)
    return out


def _fp_b(uT, kT, ufeats, kfeats, W, b2):
    Sk = kT.shape[1]
    d2 = _pair_d2(uT, kT)
    lanes = jax.lax.broadcasted_iota(jnp.int32, d2.shape, 1)
    Gs, recips = [], []
    for _ in range(3):
        m, sel = _min_extract(d2, lanes, Sk)
        dist = jnp.maximum(m, 1e-10)
        recips.append(1.0 / (dist + 1e-08))
        Gs.append(sel.astype(uT.dtype))
        d2 = jnp.where(sel, _BIG, d2)
    wsum = recips[0] + recips[1] + recips[2]
    Gw = (Gs[0] * (recips[0] / wsum) + Gs[1] * (recips[1] / wsum)
          + Gs[2] * (recips[2] / wsum))
    interp = _gather(kfeats, Gw)
    gin = interp if ufeats is None else jnp.concatenate([ufeats, interp], axis=0)
    return jax.nn.relu(jnp.dot(W, gin, preferred_element_type=jnp.float32) + b2)


def _step_vals(frameT, st1, st2, st3, W1, b1, W2, b2, W3, b3, first):
    B = frameT.shape[0]
    N = frameT.shape[2]
    S1, S2, S3 = N // 2, N // 4, N // 8

    x1, y1, z1 = _fps(frameT[:, 0], frameT[:, 1], frameT[:, 2], S1)
    xyz1 = jnp.stack([x1, y1, z1], axis=1)

    h1l, c1l = [], []
    for b in range(B):
        qT = xyz1[b]
        if first:
            sT, h, c = qT, jnp.zeros((64, S1), frameT.dtype), jnp.zeros((64, S1), frameT.dtype)
        else:
            sT, h, c = st1[0][b], st1[1][b], st1[2][b]
        hb, cb = _lstm_cell_b(qT, sT, h, c, None, W1, b1, 3 * NS, _R1SQ, 64)
        h1l.append(hb)
        c1l.append(cb)
    h1 = jnp.stack(h1l, axis=0)
    c1 = jnp.stack(c1l, axis=0)

    x2, y2, z2 = _fps(x1, y1, z1, S2)
    xyz2 = jnp.stack([x2, y2, z2], axis=1)

    h2l, c2l = [], []
    for b in range(B):
        qT = xyz2[b]
        f2 = _qgroup_b(qT, xyz1[b], h1[b], NS, _RQ2SQ)
        if first:
            sT, h, c = qT, jnp.zeros((128, S2), frameT.dtype), jnp.zeros((128, S2), frameT.dtype)
        else:
            sT, h, c = st2[0][b], st2[1][b], st2[2][b]
        hb, cb = _lstm_cell_b(qT, sT, h, c, f2, W2, b2, 2 * NS, _R2SQ, 128)
        h2l.append(hb)
        c2l.append(cb)
    h2 = jnp.stack(h2l, axis=0)
    c2 = jnp.stack(c2l, axis=0)

    x3, y3, z3 = _fps(x2, y2, z2, S3)
    xyz3 = jnp.stack([x3, y3, z3], axis=1)

    h3l, c3l = [], []
    for b in range(B):
        qT = xyz3[b]
        f3 = _qgroup_b(qT, xyz2[b], h2[b], NS, _RQ3SQ)
        if first:
            sT, h, c = qT, jnp.zeros((256, S3), frameT.dtype), jnp.zeros((256, S3), frameT.dtype)
        else:
            sT, h, c = st3[0][b], st3[1][b], st3[2][b]
        hb, cb = _lstm_cell_b(qT, sT, h, c, f3, W3, b3, NS, _R3SQ, 256)
        h3l.append(hb)
        c3l.append(cb)
    h3 = jnp.stack(h3l, axis=0)
    c3 = jnp.stack(c3l, axis=0)

    return (xyz1, h1, c1), (xyz2, h2, c2), (xyz3, h3, c3)


def _tail_vals(frameT, st1, st2, st3, Wfp3, bfp3, Wfp2, bfp2, Wfp1, bfp1,
               Wm1, bm1, Wm2, bm2):
    B = frameT.shape[0]
    xyz1, h1, _ = st1
    xyz2, h2, _ = st2
    xyz3, h3, _ = st3
    outs = []
    for b in range(B):
        l3 = _fp_b(xyz2[b], xyz3[b], h2[b], h3[b], Wfp3, bfp3)
        l2 = _fp_b(xyz1[b], xyz2[b], h1[b], l3, Wfp2, bfp2)
        l1 = _fp_b(frameT[b], xyz1[b], None, l2, Wfp1, bfp1)
        hid = jax.nn.relu(jnp.dot(Wm1, l1, preferred_element_type=jnp.float32) + bm1)
        motionT = jnp.dot(Wm2, hid, preferred_element_type=jnp.float32) + bm2
        outs.append(frameT[b] + motionT)
    return jnp.stack(outs, axis=0)


def _write_states(st, refs):
    (xyz1, h1, c1), (xyz2, h2, c2), (xyz3, h3, c3) = st
    (r_x1, r_h1, r_c1, r_x2, r_h2, r_c2, r_x3, r_h3, r_c3) = refs
    r_x1[...] = xyz1
    r_h1[...] = h1
    r_c1[...] = c1
    r_x2[...] = xyz2
    r_h2[...] = h2
    r_c2[...] = c2
    r_x3[...] = xyz3
    r_h3[...] = h3
    r_c3[...] = c3


def _first_body(frameT_ref, W1, b1, W2, b2, W3, b3, *out_refs):
    st = _step_vals(frameT_ref[...], None, None, None,
                    W1[...], b1[...], W2[...], b2[...], W3[...], b3[...],
                    first=True)
    _write_states(st, out_refs)


def _enc_body(frameT_ref, x1, h1, c1, x2, h2, c2, x3, h3, c3,
              W1, b1, W2, b2, W3, b3, *out_refs):
    st1 = (x1[...], h1[...], c1[...])
    st2 = (x2[...], h2[...], c2[...])
    st3 = (x3[...], h3[...], c3[...])
    st = _step_vals(frameT_ref[...], st1, st2, st3,
                    W1[...], b1[...], W2[...], b2[...], W3[...], b3[...],
                    first=False)
    _write_states(st, out_refs)


def _dec_body(frameT_ref, x1, h1, c1, x2, h2, c2, x3, h3, c3,
              W1, b1, W2, b2, W3, b3,
              Wfp3, bfp3, Wfp2, bfp2, Wfp1, bfp1, Wm1, bm1, Wm2, bm2,
              *out_refs):
    frameT = frameT_ref[...]
    st1 = (x1[...], h1[...], c1[...])
    st2 = (x2[...], h2[...], c2[...])
    st3 = (x3[...], h3[...], c3[...])
    st = _step_vals(frameT, st1, st2, st3,
                    W1[...], b1[...], W2[...], b2[...], W3[...], b3[...],
                    first=False)
    _write_states(st, out_refs[:9])
    new_frame = _tail_vals(frameT, st[0], st[1], st[2],
                           Wfp3[...], bfp3[...], Wfp2[...], bfp2[...],
                           Wfp1[...], bfp1[...], Wm1[...], bm1[...],
                           Wm2[...], bm2[...])
    out_refs[9][...] = new_frame


def _state_shapes(B, N, dt):
    S1, S2, S3 = N // 2, N // 4, N // 8
    return (
        jax.ShapeDtypeStruct((B, 3, S1), dt),
        jax.ShapeDtypeStruct((B, 64, S1), dt),
        jax.ShapeDtypeStruct((B, 64, S1), dt),
        jax.ShapeDtypeStruct((B, 3, S2), dt),
        jax.ShapeDtypeStruct((B, 128, S2), dt),
        jax.ShapeDtypeStruct((B, 128, S2), dt),
        jax.ShapeDtypeStruct((B, 3, S3), dt),
        jax.ShapeDtypeStruct((B, 256, S3), dt),
        jax.ShapeDtypeStruct((B, 256, S3), dt),
    )


def kernel(xyzs, Wec1, bec1, Wec2, bec2, Wec3, bec3, Wdc1, bdc1, Wdc2, bdc2,
           Wdc3, bdc3, Wfp3, bfp3, Wfp2, bfp2, Wfp1, bfp1, Wm1, bm1, Wm2, bm2):
    B, L, N, _ = xyzs.shape
    dt = xyzs.dtype
    framesT = jnp.transpose(xyzs, (0, 1, 3, 2))

    def col(v):
        return v[:, None]

    enc_w = (Wec1, col(bec1), Wec2, col(bec2), Wec3, col(bec3))
    dec_w = (Wdc1, col(bdc1), Wdc2, col(bdc2), Wdc3, col(bdc3))
    tail_w = (Wfp3, col(bfp3), Wfp2, col(bfp2), Wfp1, col(bfp1),
              Wm1, col(bm1), Wm2, col(bm2))

    st = pl.pallas_call(
        _first_body,
        out_shape=_state_shapes(B, N, dt),
    )(framesT[:, 0], *enc_w)

    for t in range(1, L // 2):
        st = pl.pallas_call(
            _enc_body,
            out_shape=_state_shapes(B, N, dt),
        )(framesT[:, t], *st, *enc_w)

    frame = framesT[:, L // 2 - 1]
    preds = []
    for _ in range(L // 2, L):
        outs = pl.pallas_call(
            _dec_body,
            out_shape=tuple(_state_shapes(B, N, dt))
            + (jax.ShapeDtypeStruct((B, 3, N), dt),),
        )(frame, *st, *dec_w, *tail_w)
        st = outs[:9]
        frame = outs[9]
        preds.append(frame)

    out = jnp.stack(preds, axis=1)
    return jnp.transpose(out, (0, 1, 3, 2))
